# baseline (speedup 1.0000x reference)
import jax
import jax.numpy as jnp
from jax import lax
from jax.experimental import pallas as pl
from jax.experimental.pallas import tpu as pltpu
from jax.experimental.pallas import tpu_sc as plsc

N = 50000
E = 800000
EB = 128
EPAD = 802816
NB = EPAD // EB
NC, NS = 2, 16
NP = 50176
G = 7
BLK = 1024
GRID = NP // BLK

_MESH = plsc.VectorSubcoreMesh(
    core_axis_name="c", subcore_axis_name="s", num_cores=NC, num_subcores=NS)


def _make_prop(d, split_edges):
  stripe = NP // NS
  zr = stripe // 4
  q = (NB // 2 if split_edges else NB) // NS
  assert q % G == 0
  rounds = q // G

  def body(x_hbm, idx_hbm, zeros_hbm, agg_hbm, shared, idxb, buf, *sems):
    gsem, ssem = sems[:G], sems[G:]
    cid = lax.axis_index("c")
    sid = lax.axis_index("s")
    for j in range(4):
      pltpu.sync_copy(zeros_hbm.at[pl.ds(0, zr)],
                      shared.at[pl.ds(sid * stripe + j * zr, zr)])
    plsc.subcore_barrier()

    if split_edges:
      tb = cid * (NB // 2) + sid * q
    else:
      tb = sid * q

    def rnd(r, c):
      b0 = tb + r * G
      if split_edges:
        pltpu.sync_copy(idx_hbm.at[pl.ds(b0, G)], idxb)
      else:
        pltpu.sync_copy(idx_hbm.at[cid, pl.ds(b0, G)], idxb)
      gds = [pltpu.async_copy(x_hbm.at[idxb.at[j, 0]], buf.at[j], gsem[j])
             for j in range(G)]
      sds = []
      for j in range(G):
        gds[j].wait()
        sds.append(pltpu.async_copy(buf.at[j], shared.at[idxb.at[j, 1]],
                                    ssem[j], add=True))
      for sd in sds:
        sd.wait()
      return c

    lax.fori_loop(0, rounds, rnd, 0)
    plsc.subcore_barrier()
    pltpu.sync_copy(shared.at[pl.ds(sid * stripe, stripe)],
                    agg_hbm.at[cid, pl.ds(sid * stripe, stripe)])

  nx = (NP, d) if split_edges else (2 * NP, d)
  prop = pl.kernel(
      body,
      out_type=jax.ShapeDtypeStruct((2, NP, d), jnp.float32),
      mesh=_MESH,
      scratch_types=[
          pltpu.VMEM_SHARED((NP, d), jnp.float32),
          pltpu.VMEM((G, 2, EB), jnp.int32),
          pltpu.VMEM((G, EB, d), jnp.float32),
      ] + [pltpu.SemaphoreType.DMA] * (2 * G),
      compiler_params=pltpu.CompilerParams(use_tc_tiling_on_sc=False),
  )

  def run(x, idx):
    assert x.shape == nx, x.shape
    zeros = jnp.zeros((zr, d), jnp.float32)
    return prop(x, idx, zeros)

  return run


_prop64 = _make_prop(32, split_edges=False)
_prop16 = _make_prop(16, split_edges=True)


M4 = NP // 4
B4 = 1568
TGRID = M4 // B4


def _leaky(h):
  return jnp.where(h >= 0, h, 0.01 * h)


def _dot(a, b):
  return jnp.dot(a, b, preferred_element_type=jnp.float32)


def _tc1_body(p_ref, sel_ref, wlo_ref, whi_ref, b_ref, xs_ref, invp_ref):
  s = p_ref[0] + p_ref[1]
  deg = _dot(s, sel_ref[...])
  invp = 1.0 / jnp.maximum(deg, 1.0)
  xp = s * invp
  xs_ref[0] = _leaky(_dot(xp, wlo_ref[...]) + b_ref[0:1])
  xs_ref[1] = _leaky(_dot(xp, whi_ref[...]) + b_ref[1:2])
  invp_ref[...] = invp


def _tcmid_body(agg_ref, invp_ref, w00_ref, w10_ref, w01_ref, w11_ref, b_ref,
                xs_ref):
  inv = invp_ref[...]
  a0 = agg_ref[0] * inv
  a1 = agg_ref[1] * inv
  xs_ref[0] = _leaky(_dot(a0, w00_ref[...]) + _dot(a1, w10_ref[...])
                     + b_ref[0:1])
  xs_ref[1] = _leaky(_dot(a0, w01_ref[...]) + _dot(a1, w11_ref[...])
                     + b_ref[1:2])


def _tc5a_body(xs_ref, w5lo_ref, w5hi_ref, t_ref):
  t_ref[...] = (_dot(xs_ref[0], w5lo_ref[...])
                + _dot(xs_ref[1], w5hi_ref[...]))


def _tc5b_body(p_ref, invp_ref, b_ref, o_ref):
  o_ref[...] = (p_ref[0] + p_ref[1]) * invp_ref[...] + b_ref[...]


def _full(shape):
  return pl.BlockSpec(shape, lambda i: tuple(0 for _ in shape))


_PK4_IN = pl.BlockSpec((2, B4, 128), lambda i: (0, i, 0))
_PK4_OUT = pl.BlockSpec((B4, 128), lambda i: (i, 0))


def _tc1(p1, sel, wlo, whi, b2r):
  return pl.pallas_call(
      _tc1_body,
      grid=(TGRID,),
      in_specs=[_PK4_IN, _full((128, 128)), _full((128, 128)),
                _full((128, 128)), _full((2, 128))],
      out_specs=[pl.BlockSpec((2, B4, 128), lambda i: (0, i, 0)), _PK4_OUT],
      out_shape=[jax.ShapeDtypeStruct((2, M4, 128), jnp.float32),
                 jax.ShapeDtypeStruct((M4, 128), jnp.float32)],
  )(p1, sel, wlo, whi, b2r)


def _tcmid(agg, invp, wbd, b2r):
  return pl.pallas_call(
      _tcmid_body,
      grid=(TGRID,),
      in_specs=[_PK4_IN, _PK4_OUT, _full((128, 128)), _full((128, 128)),
                _full((128, 128)), _full((128, 128)), _full((2, 128))],
      out_specs=pl.BlockSpec((2, B4, 128), lambda i: (0, i, 0)),
      out_shape=jax.ShapeDtypeStruct((2, M4, 128), jnp.float32),
  )(agg, invp, *wbd, b2r)


def _tc5a(xs, w5lo, w5hi):
  return pl.pallas_call(
      _tc5a_body,
      grid=(TGRID,),
      in_specs=[_PK4_IN, _full((128, 128)), _full((128, 128))],
      out_specs=_PK4_OUT,
      out_shape=jax.ShapeDtypeStruct((M4, 128), jnp.float32),
  )(xs, w5lo, w5hi)


def _tc5b(p5, invp, b5t):
  return pl.pallas_call(
      _tc5b_body,
      grid=(TGRID,),
      in_specs=[_PK4_IN, _PK4_OUT, _full((1, 128))],
      out_specs=_PK4_OUT,
      out_shape=jax.ShapeDtypeStruct((M4, 128), jnp.float32),
  )(p5, invp, b5t)


def kernel(features, edge_index, W1, b1, W2, b2, W3, b3, W4, b4, W5, b5):
  f32 = jnp.float32
  i32 = jnp.int32
  eye4 = jnp.eye(4, dtype=f32)

  ei = edge_index.astype(i32)
  npad = EPAD - E
  src = jnp.concatenate([ei[0], jnp.zeros((npad,), i32)]).reshape(NB, EB)
  dst = jnp.concatenate(
      [ei[1], N + jnp.arange(npad, dtype=i32) % (NP - N)]).reshape(NB, EB)
  idx16 = jnp.stack([src, dst], axis=1)
  idx64 = jnp.stack([idx16, jnp.stack([src + NP, dst], axis=1)])

  x16 = jnp.concatenate(
      [features.astype(f32), jnp.ones((N, 1), f32), jnp.zeros((N, 11), f32)],
      axis=1)
  x16 = jnp.pad(x16, ((0, NP - N), (0, 0)))

  sel = jnp.kron(eye4, jnp.zeros((32, 32), f32).at[4, :].set(1.0))

  w1p = jnp.zeros((32, 64), f32).at[0:4, :].set(W1)
  w1lo = jnp.kron(eye4, w1p[:, 0:32])
  w1hi = jnp.kron(eye4, w1p[:, 32:64])
  b1t = jnp.stack([jnp.tile(b1[0:32], 4), jnp.tile(b1[32:64], 4)])

  p1 = jnp.pad(_prop16(x16, idx16), ((0, 0), (0, 0), (0, 16)))
  p1 = p1.reshape(2, M4, 128)
  xs, invp = _tc1(p1, sel, w1lo, w1hi, b1t)

  for w, b in ((W2, b2), (W3, b3), (W4, b4)):
    wbd = [jnp.kron(eye4, w[0:32, 0:32]), jnp.kron(eye4, w[32:64, 0:32]),
           jnp.kron(eye4, w[0:32, 32:64]), jnp.kron(eye4, w[32:64, 32:64])]
    b2r = jnp.stack([jnp.tile(b[0:32], 4), jnp.tile(b[32:64], 4)])
    agg = _prop64(xs.reshape(2 * NP, 32), idx64).reshape(2, M4, 128)
    xs = _tcmid(agg, invp, wbd, b2r)

  w5p = jnp.zeros((64, 32), f32).at[:, 0:3].set(W5)
  w5lo = jnp.kron(eye4, w5p[0:32])
  w5hi = jnp.kron(eye4, w5p[32:64])
  t16 = _tc5a(xs, w5lo, w5hi).reshape(NP, 32)[:, 0:16]
  p5 = jnp.pad(_prop16(t16, idx16), ((0, 0), (0, 0), (0, 16)))
  p5 = p5.reshape(2, M4, 128)
  b5t = jnp.tile(jnp.pad(b5, (0, 29)), 4).reshape(1, 128)
  out32 = _tc5b(p5, invp, b5t).reshape(NP, 32)
  return out32[:N, :3]


# speedup vs baseline: 1.1139x; 1.1139x over previous
import jax
import jax.numpy as jnp
from jax import lax
from jax.experimental import pallas as pl
from jax.experimental.pallas import tpu as pltpu
from jax.experimental.pallas import tpu_sc as plsc

N = 50000
E = 800000
EB = 128
EPAD = 802816
NB = EPAD // EB
NC, NS = 2, 16
NP = 50176
G = 7
BLK = 1024
GRID = NP // BLK

_MESH = plsc.VectorSubcoreMesh(
    core_axis_name="c", subcore_axis_name="s", num_cores=NC, num_subcores=NS)


def _make_prop(d, split_edges):
  stripe = NP // NS
  zr = stripe // 4
  q = (NB // 2 if split_edges else NB) // NS
  assert q % G == 0
  rounds = q // G

  def body(x_hbm, src_hbm, dst_hbm, zeros_hbm, agg_hbm,
           shared, sidxb, didxb, buf, *sems):
    gsem, ssem = sems[:G], sems[G:]
    cid = lax.axis_index("c")
    sid = lax.axis_index("s")
    for j in range(4):
      pltpu.sync_copy(zeros_hbm.at[pl.ds(0, zr)],
                      shared.at[pl.ds(sid * stripe + j * zr, zr)])
    plsc.subcore_barrier()

    if split_edges:
      tb = cid * (NB // 2) + sid * q
      srow = 0
    else:
      tb = sid * q
      srow = cid

    pltpu.sync_copy(src_hbm.at[srow, pl.ds(tb, G)], sidxb)

    def rnd(r, c):
      b0 = tb + r * G
      pltpu.sync_copy(dst_hbm.at[pl.ds(b0, G)], didxb)
      gds = [pltpu.async_copy(x_hbm.at[sidxb.at[j]], buf.at[j], gsem[j])
             for j in range(G)]
      sds = []
      for j in range(G):
        gds[j].wait()
        sds.append(pltpu.async_copy(buf.at[j], shared.at[didxb.at[j]],
                                    ssem[j], add=True))
      bn = jnp.minimum(b0 + G, NB - G)
      pltpu.sync_copy(src_hbm.at[srow, pl.ds(bn, G)], sidxb)
      for sd in sds:
        sd.wait()
      return c

    lax.fori_loop(0, rounds, rnd, 0)
    plsc.subcore_barrier()
    pltpu.sync_copy(shared.at[pl.ds(sid * stripe, stripe)],
                    agg_hbm.at[cid, pl.ds(sid * stripe, stripe)])

  nx = (NP, d) if split_edges else (2 * NP, d)
  prop = pl.kernel(
      body,
      out_type=jax.ShapeDtypeStruct((2, NP, d), jnp.float32),
      mesh=_MESH,
      scratch_types=[
          pltpu.VMEM_SHARED((NP, d), jnp.float32),
          pltpu.VMEM((G, EB), jnp.int32),
          pltpu.VMEM((G, EB), jnp.int32),
          pltpu.VMEM((G, EB, d), jnp.float32),
      ] + [pltpu.SemaphoreType.DMA] * (2 * G),
      compiler_params=pltpu.CompilerParams(use_tc_tiling_on_sc=False),
  )

  def run(x, src2, dstb):
    assert x.shape == nx, x.shape
    zeros = jnp.zeros((zr, d), jnp.float32)
    return prop(x, src2, dstb, zeros)

  return run


_prop64 = _make_prop(32, split_edges=False)
_prop32 = _make_prop(32, split_edges=True)


M4 = NP // 4
B4 = 1568
TGRID = M4 // B4


def _leaky(h):
  return jnp.where(h >= 0, h, 0.01 * h)


def _dot(a, b):
  return jnp.dot(a, b, preferred_element_type=jnp.float32)


def _tc1_body(p_ref, sel_ref, wlo_ref, whi_ref, b_ref, xs_ref, invp_ref):
  s = p_ref[0] + p_ref[1]
  deg = _dot(s, sel_ref[...])
  invp = 1.0 / jnp.maximum(deg, 1.0)
  xp = s * invp
  xs_ref[0] = _leaky(_dot(xp, wlo_ref[...]) + b_ref[0:1])
  xs_ref[1] = _leaky(_dot(xp, whi_ref[...]) + b_ref[1:2])
  invp_ref[...] = invp


def _tcmid_body(agg_ref, invp_ref, w00_ref, w10_ref, w01_ref, w11_ref, b_ref,
                xs_ref):
  inv = invp_ref[...]
  a0 = agg_ref[0] * inv
  a1 = agg_ref[1] * inv
  xs_ref[0] = _leaky(_dot(a0, w00_ref[...]) + _dot(a1, w10_ref[...])
                     + b_ref[0:1])
  xs_ref[1] = _leaky(_dot(a0, w01_ref[...]) + _dot(a1, w11_ref[...])
                     + b_ref[1:2])


def _tc5a_body(xs_ref, w5lo_ref, w5hi_ref, t_ref):
  t_ref[...] = (_dot(xs_ref[0], w5lo_ref[...])
                + _dot(xs_ref[1], w5hi_ref[...]))


def _tc5b_body(p_ref, invp_ref, b_ref, o_ref):
  o_ref[...] = (p_ref[0] + p_ref[1]) * invp_ref[...] + b_ref[...]


def _full(shape):
  return pl.BlockSpec(shape, lambda i: tuple(0 for _ in shape))


_PK4_IN = pl.BlockSpec((2, B4, 128), lambda i: (0, i, 0))
_PK4_OUT = pl.BlockSpec((B4, 128), lambda i: (i, 0))


def _tc1(p1, sel, wlo, whi, b2r):
  return pl.pallas_call(
      _tc1_body,
      grid=(TGRID,),
      in_specs=[_PK4_IN, _full((128, 128)), _full((128, 128)),
                _full((128, 128)), _full((2, 128))],
      out_specs=[pl.BlockSpec((2, B4, 128), lambda i: (0, i, 0)), _PK4_OUT],
      out_shape=[jax.ShapeDtypeStruct((2, M4, 128), jnp.float32),
                 jax.ShapeDtypeStruct((M4, 128), jnp.float32)],
  )(p1, sel, wlo, whi, b2r)


def _tcmid(agg, invp, wbd, b2r):
  return pl.pallas_call(
      _tcmid_body,
      grid=(TGRID,),
      in_specs=[_PK4_IN, _PK4_OUT, _full((128, 128)), _full((128, 128)),
                _full((128, 128)), _full((128, 128)), _full((2, 128))],
      out_specs=pl.BlockSpec((2, B4, 128), lambda i: (0, i, 0)),
      out_shape=jax.ShapeDtypeStruct((2, M4, 128), jnp.float32),
  )(agg, invp, *wbd, b2r)


def _tc5a(xs, w5lo, w5hi):
  return pl.pallas_call(
      _tc5a_body,
      grid=(TGRID,),
      in_specs=[_PK4_IN, _full((128, 128)), _full((128, 128))],
      out_specs=_PK4_OUT,
      out_shape=jax.ShapeDtypeStruct((M4, 128), jnp.float32),
  )(xs, w5lo, w5hi)


def _tc5b(p5, invp, b5t):
  return pl.pallas_call(
      _tc5b_body,
      grid=(TGRID,),
      in_specs=[_PK4_IN, _PK4_OUT, _full((1, 128))],
      out_specs=_PK4_OUT,
      out_shape=jax.ShapeDtypeStruct((M4, 128), jnp.float32),
  )(p5, invp, b5t)


def kernel(features, edge_index, W1, b1, W2, b2, W3, b3, W4, b4, W5, b5):
  f32 = jnp.float32
  i32 = jnp.int32
  eye4 = jnp.eye(4, dtype=f32)

  ei = edge_index.astype(i32)
  npad = EPAD - E
  src = jnp.concatenate([ei[0], jnp.zeros((npad,), i32)]).reshape(NB, EB)
  dst = jnp.concatenate(
      [ei[1], N + jnp.arange(npad, dtype=i32) % (NP - N)]).reshape(NB, EB)
  src2 = jnp.stack([src, src + NP])

  x32 = jnp.concatenate(
      [features.astype(f32), jnp.ones((N, 1), f32), jnp.zeros((N, 27), f32)],
      axis=1)
  x32 = jnp.pad(x32, ((0, NP - N), (0, 0)))

  sel = jnp.kron(eye4, jnp.zeros((32, 32), f32).at[4, :].set(1.0))

  w1p = jnp.zeros((32, 64), f32).at[0:4, :].set(W1)
  w1lo = jnp.kron(eye4, w1p[:, 0:32])
  w1hi = jnp.kron(eye4, w1p[:, 32:64])
  b1t = jnp.stack([jnp.tile(b1[0:32], 4), jnp.tile(b1[32:64], 4)])

  p1 = _prop32(x32, src2, dst).reshape(2, M4, 128)
  xs, invp = _tc1(p1, sel, w1lo, w1hi, b1t)

  for w, b in ((W2, b2), (W3, b3), (W4, b4)):
    wbd = [jnp.kron(eye4, w[0:32, 0:32]), jnp.kron(eye4, w[32:64, 0:32]),
           jnp.kron(eye4, w[0:32, 32:64]), jnp.kron(eye4, w[32:64, 32:64])]
    b2r = jnp.stack([jnp.tile(b[0:32], 4), jnp.tile(b[32:64], 4)])
    agg = _prop64(xs.reshape(2 * NP, 32), src2, dst).reshape(2, M4, 128)
    xs = _tcmid(agg, invp, wbd, b2r)

  w5p = jnp.zeros((64, 32), f32).at[:, 0:3].set(W5)
  w5lo = jnp.kron(eye4, w5p[0:32])
  w5hi = jnp.kron(eye4, w5p[32:64])
  t32 = _tc5a(xs, w5lo, w5hi)
  p5 = _prop32(t32.reshape(NP, 32), src2, dst).reshape(2, M4, 128)
  b5t = jnp.tile(jnp.pad(b5, (0, 29)), 4).reshape(1, 128)
  out32 = _tc5b(p5, invp, b5t).reshape(NP, 32)
  return out32[:N, :3]


# speedup vs baseline: 1.2185x; 1.0938x over previous
import jax
import jax.numpy as jnp
from jax import lax
from jax.experimental import pallas as pl
from jax.experimental.pallas import tpu as pltpu
from jax.experimental.pallas import tpu_sc as plsc

N = 50000
E = 800000
EB = 128
EPAD = 802816
NB = EPAD // EB
NC, NS = 2, 16
NP = 50176
G = 7
BLK = 1024
GRID = NP // BLK

_MESH = plsc.VectorSubcoreMesh(
    core_axis_name="c", subcore_axis_name="s", num_cores=NC, num_subcores=NS)


def _make_prop(d, split_edges):
  stripe = NP // NS
  zr = stripe // 4
  q = (NB // 2 if split_edges else NB) // NS
  assert q % G == 0
  rounds = q // G

  def body(x_hbm, src_hbm, dst_hbm, zeros_hbm, agg_hbm,
           shared, sidxb, didxb, buf, *sems):
    gsem, ssem = sems[:G], sems[G:]
    cid = lax.axis_index("c")
    sid = lax.axis_index("s")
    for j in range(4):
      pltpu.sync_copy(zeros_hbm.at[pl.ds(0, zr)],
                      shared.at[pl.ds(sid * stripe + j * zr, zr)])
    plsc.subcore_barrier()

    if split_edges:
      tb = cid * (NB // 2) + sid * q
      srow = 0
    else:
      tb = sid * q
      srow = cid

    pltpu.sync_copy(src_hbm.at[srow, pl.ds(tb, G)], sidxb)

    def rnd(r, c):
      b0 = tb + r * G
      gds = [pltpu.async_copy(x_hbm.at[sidxb.at[j]], buf.at[j], gsem[j])
             for j in range(G)]
      pltpu.sync_copy(dst_hbm.at[pl.ds(b0, G)], didxb)
      sds = []
      for j in range(G):
        gds[j].wait()
        sds.append(pltpu.async_copy(buf.at[j], shared.at[didxb.at[j]],
                                    ssem[j], add=True))
      bn = jnp.minimum(b0 + G, NB - G)
      pltpu.sync_copy(src_hbm.at[srow, pl.ds(bn, G)], sidxb)
      for sd in sds:
        sd.wait()
      return c

    lax.fori_loop(0, rounds, rnd, 0)
    plsc.subcore_barrier()
    pltpu.sync_copy(shared.at[pl.ds(sid * stripe, stripe)],
                    agg_hbm.at[cid, pl.ds(sid * stripe, stripe)])

  nx = (NP, d) if split_edges else (2 * NP, d)
  prop = pl.kernel(
      body,
      out_type=jax.ShapeDtypeStruct((2, NP, d), jnp.float32),
      mesh=_MESH,
      scratch_types=[
          pltpu.VMEM_SHARED((NP, d), jnp.float32),
          pltpu.VMEM((G, EB), jnp.int32),
          pltpu.VMEM((G, EB), jnp.int32),
          pltpu.VMEM((G, EB, d), jnp.float32),
      ] + [pltpu.SemaphoreType.DMA] * (2 * G),
      compiler_params=pltpu.CompilerParams(use_tc_tiling_on_sc=False),
  )

  def run(x, src2, dstb):
    assert x.shape == nx, x.shape
    zeros = jnp.zeros((zr, d), jnp.float32)
    return prop(x, src2, dstb, zeros)

  return run


_prop64 = _make_prop(32, split_edges=False)
_prop32 = _make_prop(32, split_edges=True)


M4 = NP // 4
B4 = 1568
TGRID = M4 // B4


def _leaky(h):
  return jnp.where(h >= 0, h, 0.01 * h)


def _dot(a, b):
  return jnp.dot(a, b, preferred_element_type=jnp.float32)


def _tc1_body(p_ref, sel_ref, wlo_ref, whi_ref, b_ref, xs_ref, invp_ref):
  s = p_ref[0] + p_ref[1]
  deg = _dot(s, sel_ref[...])
  invp = 1.0 / jnp.maximum(deg, 1.0)
  xp = s * invp
  xs_ref[0] = _leaky(_dot(xp, wlo_ref[...]) + b_ref[0:1])
  xs_ref[1] = _leaky(_dot(xp, whi_ref[...]) + b_ref[1:2])
  invp_ref[...] = invp


def _tcmid_body(agg_ref, invp_ref, w00_ref, w10_ref, w01_ref, w11_ref, b_ref,
                xs_ref):
  inv = invp_ref[...]
  a0 = agg_ref[0] * inv
  a1 = agg_ref[1] * inv
  xs_ref[0] = _leaky(_dot(a0, w00_ref[...]) + _dot(a1, w10_ref[...])
                     + b_ref[0:1])
  xs_ref[1] = _leaky(_dot(a0, w01_ref[...]) + _dot(a1, w11_ref[...])
                     + b_ref[1:2])


def _tc5a_body(xs_ref, w5lo_ref, w5hi_ref, t_ref):
  t_ref[...] = (_dot(xs_ref[0], w5lo_ref[...])
                + _dot(xs_ref[1], w5hi_ref[...]))


def _tc5b_body(p_ref, invp_ref, b_ref, o_ref):
  o_ref[...] = (p_ref[0] + p_ref[1]) * invp_ref[...] + b_ref[...]


def _full(shape):
  return pl.BlockSpec(shape, lambda i: tuple(0 for _ in shape))


_PK4_IN = pl.BlockSpec((2, B4, 128), lambda i: (0, i, 0))
_PK4_OUT = pl.BlockSpec((B4, 128), lambda i: (i, 0))


def _tc1(p1, sel, wlo, whi, b2r):
  return pl.pallas_call(
      _tc1_body,
      grid=(TGRID,),
      in_specs=[_PK4_IN, _full((128, 128)), _full((128, 128)),
                _full((128, 128)), _full((2, 128))],
      out_specs=[pl.BlockSpec((2, B4, 128), lambda i: (0, i, 0)), _PK4_OUT],
      out_shape=[jax.ShapeDtypeStruct((2, M4, 128), jnp.float32),
                 jax.ShapeDtypeStruct((M4, 128), jnp.float32)],
  )(p1, sel, wlo, whi, b2r)


def _tcmid(agg, invp, wbd, b2r):
  return pl.pallas_call(
      _tcmid_body,
      grid=(TGRID,),
      in_specs=[_PK4_IN, _PK4_OUT, _full((128, 128)), _full((128, 128)),
                _full((128, 128)), _full((128, 128)), _full((2, 128))],
      out_specs=pl.BlockSpec((2, B4, 128), lambda i: (0, i, 0)),
      out_shape=jax.ShapeDtypeStruct((2, M4, 128), jnp.float32),
  )(agg, invp, *wbd, b2r)


def _tc5a(xs, w5lo, w5hi):
  return pl.pallas_call(
      _tc5a_body,
      grid=(TGRID,),
      in_specs=[_PK4_IN, _full((128, 128)), _full((128, 128))],
      out_specs=_PK4_OUT,
      out_shape=jax.ShapeDtypeStruct((M4, 128), jnp.float32),
  )(xs, w5lo, w5hi)


def _tc5b(p5, invp, b5t):
  return pl.pallas_call(
      _tc5b_body,
      grid=(TGRID,),
      in_specs=[_PK4_IN, _PK4_OUT, _full((1, 128))],
      out_specs=_PK4_OUT,
      out_shape=jax.ShapeDtypeStruct((M4, 128), jnp.float32),
  )(p5, invp, b5t)


def kernel(features, edge_index, W1, b1, W2, b2, W3, b3, W4, b4, W5, b5):
  f32 = jnp.float32
  i32 = jnp.int32
  eye4 = jnp.eye(4, dtype=f32)

  ei = edge_index.astype(i32)
  npad = EPAD - E
  src = jnp.concatenate([ei[0], jnp.zeros((npad,), i32)]).reshape(NB, EB)
  dst = jnp.concatenate(
      [ei[1], N + jnp.arange(npad, dtype=i32) % (NP - N)]).reshape(NB, EB)
  src2 = jnp.stack([src, src + NP])

  x32 = jnp.concatenate(
      [features.astype(f32), jnp.ones((N, 1), f32), jnp.zeros((N, 27), f32)],
      axis=1)
  x32 = jnp.pad(x32, ((0, NP - N), (0, 0)))

  sel = jnp.kron(eye4, jnp.zeros((32, 32), f32).at[4, :].set(1.0))

  w1p = jnp.zeros((32, 64), f32).at[0:4, :].set(W1)
  w1lo = jnp.kron(eye4, w1p[:, 0:32])
  w1hi = jnp.kron(eye4, w1p[:, 32:64])
  b1t = jnp.stack([jnp.tile(b1[0:32], 4), jnp.tile(b1[32:64], 4)])

  p1 = _prop32(x32, src2, dst).reshape(2, M4, 128)
  xs, invp = _tc1(p1, sel, w1lo, w1hi, b1t)

  for w, b in ((W2, b2), (W3, b3), (W4, b4)):
    wbd = [jnp.kron(eye4, w[0:32, 0:32]), jnp.kron(eye4, w[32:64, 0:32]),
           jnp.kron(eye4, w[0:32, 32:64]), jnp.kron(eye4, w[32:64, 32:64])]
    b2r = jnp.stack([jnp.tile(b[0:32], 4), jnp.tile(b[32:64], 4)])
    agg = _prop64(xs.reshape(2 * NP, 32), src2, dst).reshape(2, M4, 128)
    xs = _tcmid(agg, invp, wbd, b2r)

  w5p = jnp.zeros((64, 32), f32).at[:, 0:3].set(W5)
  w5lo = jnp.kron(eye4, w5p[0:32])
  w5hi = jnp.kron(eye4, w5p[32:64])
  t32 = _tc5a(xs, w5lo, w5hi)
  p5 = _prop32(t32.reshape(NP, 32), src2, dst).reshape(2, M4, 128)
  b5t = jnp.tile(jnp.pad(b5, (0, 29)), 4).reshape(1, 128)
  out32 = _tc5b(p5, invp, b5t).reshape(NP, 32)
  return out32[:N, :3]
